# baseline (speedup 1.0000x reference)
import functools

import jax
import jax.numpy as jnp
from jax import lax
from jax.experimental import pallas as pl
from jax.experimental.pallas import tpu as pltpu
from jax.experimental.pallas import tpu_sc as plsc

F = 26
V = 40000
B = 16384
L = 16
H = B // 2
R = H // 16
TAB = 40064
UNROLL = 8


def _fm_body(xt_hbm, wt_hbm, out_hbm,
             taba_v, tabb_v, fidxa_v, fidxb_v, local_v, red_v, acc_v, sums_sh,
             sem_ta, sem_tb, sem_ia, sem_ib, sem_r):
    cid = lax.axis_index("c")
    sid = lax.axis_index("s")
    tabs = (taba_v, tabb_v)
    fidxs = (fidxa_v, fidxb_v)
    sems = ((sem_ta, sem_ia), (sem_tb, sem_ib))

    def issue(k):
        gf = sid + 16 * k

        @pl.when(gf < F)
        def _():
            d = (gf % 2) * 64
            r0 = pl.multiple_of(gf * V - d, 128)
            pltpu.async_copy(wt_hbm.at[0, pl.ds(r0, TAB)], tabs[k], sems[k][0])
            pltpu.async_copy(xt_hbm.at[gf, pl.ds(cid * H, H)], fidxs[k], sems[k][1])

    issue(0)

    for k in range(2):
        gf = sid + 16 * k

        @pl.when(gf < F)
        def _():
            d = (gf % 2) * 64
            pltpu.make_async_copy(wt_hbm.at[0, pl.ds(0, TAB)], tabs[k], sems[k][0]).wait()
            pltpu.make_async_copy(xt_hbm.at[0, pl.ds(0, H)], fidxs[k], sems[k][1]).wait()
            tab_v = tabs[k]
            fidx_v = fidxs[k]

        if k == 0:
            issue(1)

        @pl.when(gf < F)
        def _():
            d = (gf % 2) * 64
            tab_v = tabs[k]
            fidx_v = fidxs[k]

            @plsc.parallel_loop(0, H // L, unroll=UNROLL)
            def gather_chunk(c):
                o = c * L
                idx = fidx_v[pl.ds(o, L)] + d
                g = plsc.load_gather(tab_v, [idx])
                if k == 0:
                    local_v[pl.ds(o, L)] = g
                else:
                    local_v[pl.ds(o, L)] = local_v[pl.ds(o, L)] + g

    pltpu.sync_copy(local_v, sums_sh.at[sid])
    plsc.subcore_barrier()

    row_copies = [
        pltpu.async_copy(sums_sh.at[j, pl.ds(sid * R, R)], red_v.at[j], sem_r)
        for j in range(16)
    ]
    for c in row_copies:
        c.wait()

    @plsc.parallel_loop(0, R // L, unroll=2)
    def add_chunk(c):
        o = c * L
        acc = red_v[0, pl.ds(o, L)]
        for j in range(1, 16):
            acc = acc + red_v[j, pl.ds(o, L)]
        acc_v[pl.ds(o, L)] = acc

    pltpu.sync_copy(acc_v, out_hbm.at[pl.ds(cid * H + sid * R, R)])


_fm_sc = functools.partial(
    pl.kernel,
    out_type=jax.ShapeDtypeStruct((B,), jnp.float32),
    mesh=plsc.VectorSubcoreMesh(core_axis_name="c", subcore_axis_name="s"),
    compiler_params=pltpu.CompilerParams(needs_layout_passes=False),
    scratch_types=[
        pltpu.VMEM((TAB,), jnp.float32),
        pltpu.VMEM((TAB,), jnp.float32),
        pltpu.VMEM((H,), jnp.int32),
        pltpu.VMEM((H,), jnp.int32),
        pltpu.VMEM((H,), jnp.float32),
        pltpu.VMEM((16, R), jnp.float32),
        pltpu.VMEM((R,), jnp.float32),
        pltpu.VMEM_SHARED((16, H), jnp.float32),
        pltpu.SemaphoreType.DMA,
        pltpu.SemaphoreType.DMA,
        pltpu.SemaphoreType.DMA,
        pltpu.SemaphoreType.DMA,
        pltpu.SemaphoreType.DMA,
    ],
)(_fm_body)


@jax.jit
def kernel(x, W):
    out = _fm_sc(x.T, W.T)
    return out.reshape(B, 1)


# speedup vs baseline: 1.0066x; 1.0066x over previous
import functools

import jax
import jax.numpy as jnp
from jax import lax
from jax.experimental import pallas as pl
from jax.experimental.pallas import tpu as pltpu
from jax.experimental.pallas import tpu_sc as plsc

F = 26
V = 40000
B = 16384
L = 16
H = B // 2
R = H // 16
TAB = 40064
UNROLL = 8


def _fm_body(xt_hbm, wt_hbm, out_hbm,
             taba_v, tabb_v, fidxa_v, fidxb_v, local_v, red_v, acc_v, sums_sh,
             sem_ta, sem_tb, sem_ia, sem_ib, sem_r):
    cid = lax.axis_index("c")
    sid = lax.axis_index("s")
    tabs = (taba_v, tabb_v)
    fidxs = (fidxa_v, fidxb_v)
    sems = ((sem_ta, sem_ia), (sem_tb, sem_ib))

    for k in range(2):
        gf = sid + 16 * k

        @pl.when(gf < F)
        def _():
            d = (gf % 2) * 64
            r0 = pl.multiple_of(gf * V - d, 128)
            pltpu.async_copy(wt_hbm.at[0, pl.ds(r0, TAB)], tabs[k], sems[k][0])
            pltpu.async_copy(xt_hbm.at[gf, pl.ds(cid * H, H)], fidxs[k], sems[k][1])

    for k in range(2):
        gf = sid + 16 * k

        @pl.when(gf < F)
        def _():
            d = (gf % 2) * 64
            pltpu.make_async_copy(wt_hbm.at[0, pl.ds(0, TAB)], tabs[k], sems[k][0]).wait()
            pltpu.make_async_copy(xt_hbm.at[0, pl.ds(0, H)], fidxs[k], sems[k][1]).wait()
            tab_v = tabs[k]
            fidx_v = fidxs[k]

            @plsc.parallel_loop(0, H // L, unroll=UNROLL)
            def gather_chunk(c):
                o = c * L
                idx = fidx_v[pl.ds(o, L)] + d
                g = plsc.load_gather(tab_v, [idx])
                if k == 0:
                    local_v[pl.ds(o, L)] = g
                else:
                    local_v[pl.ds(o, L)] = local_v[pl.ds(o, L)] + g

    pltpu.sync_copy(local_v, sums_sh.at[sid])
    plsc.subcore_barrier()

    row_copies = [
        pltpu.async_copy(sums_sh.at[j, pl.ds(sid * R, R)], red_v.at[j], sem_r)
        for j in range(16)
    ]
    for c in row_copies:
        c.wait()

    @plsc.parallel_loop(0, R // L, unroll=2)
    def add_chunk(c):
        o = c * L
        acc = red_v[0, pl.ds(o, L)]
        for j in range(1, 16):
            acc = acc + red_v[j, pl.ds(o, L)]
        acc_v[pl.ds(o, L)] = acc

    pltpu.sync_copy(acc_v, out_hbm.at[pl.ds(cid * H + sid * R, R)])


_fm_sc = functools.partial(
    pl.kernel,
    out_type=jax.ShapeDtypeStruct((B,), jnp.float32),
    mesh=plsc.VectorSubcoreMesh(core_axis_name="c", subcore_axis_name="s"),
    compiler_params=pltpu.CompilerParams(needs_layout_passes=False),
    scratch_types=[
        pltpu.VMEM((TAB,), jnp.float32),
        pltpu.VMEM((TAB,), jnp.float32),
        pltpu.VMEM((H,), jnp.int32),
        pltpu.VMEM((H,), jnp.int32),
        pltpu.VMEM((H,), jnp.float32),
        pltpu.VMEM((16, R), jnp.float32),
        pltpu.VMEM((R,), jnp.float32),
        pltpu.VMEM_SHARED((16, H), jnp.float32),
        pltpu.SemaphoreType.DMA,
        pltpu.SemaphoreType.DMA,
        pltpu.SemaphoreType.DMA,
        pltpu.SemaphoreType.DMA,
        pltpu.SemaphoreType.DMA,
    ],
)(_fm_body)


@jax.jit
def kernel(x, W):
    out = _fm_sc(x.T, W.T)
    return out.reshape(B, 1)
